# speedup vs baseline: 1.0581x; 1.0581x over previous
import functools

import jax
import jax.numpy as jnp
from jax import lax
from jax.experimental import pallas as pl
from jax.experimental.pallas import tpu as pltpu
from jax.experimental.pallas import tpu_sc as plsc

_BATCH = 16384
_LANES = 16
_CHUNK = 128
_NW = 32
_ROWS_PER_W = _BATCH // (_NW * _CHUNK)
_NROWS = _BATCH // _CHUNK


def _sc_kernel(users_hbm, items_hbm, ubias_hbm, ibias_hbm, out_hbm,
               uidx_v, iidx_v, urow_v, irow_v, sem_u, sem_i):
    wid = lax.axis_index("s") * 2 + lax.axis_index("c")
    base = wid * _ROWS_PER_W

    pltpu.sync_copy(users_hbm.at[pl.ds(base, _ROWS_PER_W)], uidx_v)
    pltpu.sync_copy(items_hbm.at[pl.ds(base, _ROWS_PER_W)], iidx_v)

    copies = []
    for j in range(_ROWS_PER_W):
        copies.append(pltpu.async_copy(ubias_hbm.at[uidx_v.at[j]],
                                       urow_v.at[j], sem_u))
        copies.append(pltpu.async_copy(ibias_hbm.at[iidx_v.at[j]],
                                       irow_v.at[j], sem_i))
    for c in copies:
        c.wait()

    for j in range(_ROWS_PER_W):
        for k in range(_CHUNK // _LANES):
            s = pl.ds(k * _LANES, _LANES)
            urow_v[j, s] = urow_v[j, s] * irow_v[j, s]

    pltpu.sync_copy(urow_v, out_hbm.at[pl.ds(base, _ROWS_PER_W)])


@jax.jit
def _run(users2d, items2d, ubias_flat, ibias_flat):
    mesh = plsc.VectorSubcoreMesh(core_axis_name="c", subcore_axis_name="s")
    f = functools.partial(
        pl.kernel,
        mesh=mesh,
        out_type=jax.ShapeDtypeStruct((_NROWS, _CHUNK), jnp.float32),
        scratch_types=[
            pltpu.VMEM((_ROWS_PER_W, _CHUNK), jnp.int32),
            pltpu.VMEM((_ROWS_PER_W, _CHUNK), jnp.int32),
            pltpu.VMEM((_ROWS_PER_W, _CHUNK), jnp.float32),
            pltpu.VMEM((_ROWS_PER_W, _CHUNK), jnp.float32),
            pltpu.SemaphoreType.DMA,
            pltpu.SemaphoreType.DMA,
        ],
    )(_sc_kernel)
    return f(users2d, items2d, ubias_flat, ibias_flat)


def kernel(users, items, user_bias, item_bias):
    users2d = users.astype(jnp.int32).reshape(_NROWS, _CHUNK)
    items2d = items.astype(jnp.int32).reshape(_NROWS, _CHUNK)
    out = _run(users2d, items2d, user_bias.reshape(-1), item_bias.reshape(-1))
    return out.reshape(-1)
